# Initial kernel scaffold; baseline (speedup 1.0000x reference)
import functools

import jax
import jax.numpy as jnp
from jax import lax
from jax.experimental import pallas as pl
from jax.experimental.pallas import tpu as pltpu
from jax.experimental.pallas import tpu_sc as plsc

NC = 2
NS = 16
NW = NC * NS
CH = 100


def _sc_segsum(table, gidx, sidx, zeros):
  R, D = table.shape
  nchunk = gidx.shape[1]
  rpt = R // NS
  mesh = plsc.VectorSubcoreMesh(core_axis_name="c", subcore_axis_name="s")

  @functools.partial(
      pl.kernel,
      out_type=jax.ShapeDtypeStruct((NC * R, D), jnp.float32),
      mesh=mesh,
      scratch_types=[
          pltpu.VMEM((nchunk, CH), jnp.int32),
          pltpu.VMEM((nchunk, CH), jnp.int32),
          pltpu.VMEM((CH, D), jnp.float32),
          pltpu.VMEM((CH, D), jnp.float32),
          pltpu.VMEM_SHARED((R, D), jnp.float32),
          pltpu.SemaphoreType.DMA,
          pltpu.SemaphoreType.DMA,
      ],
  )
  def k(table_hbm, gidx_hbm, sidx_hbm, zeros_hbm, part_hbm,
        gidx_v, sidx_v, rows0, rows1, acc, sem0, sem1):
    c = lax.axis_index("c")
    s = lax.axis_index("s")
    wid = s * NC + c
    pltpu.sync_copy(zeros_hbm.at[pl.ds(s * rpt, rpt)], acc.at[pl.ds(s * rpt, rpt)])
    pltpu.sync_copy(gidx_hbm.at[wid], gidx_v)
    pltpu.sync_copy(sidx_hbm.at[wid], sidx_v)
    plsc.subcore_barrier()

    pltpu.make_async_copy(table_hbm.at[gidx_v.at[0]], rows0, sem0).start()

    @pl.loop(0, nchunk // 2)
    def _(kk):
      g0 = 2 * kk
      pltpu.make_async_copy(table_hbm.at[gidx_v.at[g0 + 1]], rows1, sem1).start()
      pltpu.make_async_copy(table_hbm.at[gidx_v.at[g0]], rows0, sem0).wait()
      pltpu.sync_copy(rows0, acc.at[sidx_v.at[g0]], add=True)

      @pl.when(kk < nchunk // 2 - 1)
      def _():
        pltpu.make_async_copy(table_hbm.at[gidx_v.at[g0 + 2]], rows0, sem0).start()

      pltpu.make_async_copy(table_hbm.at[gidx_v.at[g0 + 1]], rows1, sem1).wait()
      pltpu.sync_copy(rows1, acc.at[sidx_v.at[g0 + 1]], add=True)

    plsc.subcore_barrier()
    pltpu.sync_copy(acc.at[pl.ds(s * rpt, rpt)],
                    part_hbm.at[pl.ds(c * R + s * rpt, rpt)])

  return k(table, gidx, sidx, zeros)


def _tc_phase1(vfeat, vrw, W1, Wve, b1, bve):
  n, d_in = vfeat.shape
  de = Wve.shape[1]
  bn = 2000

  def body(vf, wr, w1, wv, b1r, bver, o):
    t = jnp.dot(vf[...], w1[...], preferred_element_type=jnp.float32)
    bias = jnp.dot(b1r[...], wv[...], preferred_element_type=jnp.float32) + bver[...]
    u = jnp.dot(t, wv[...], preferred_element_type=jnp.float32) + bias
    o[...] = wr[...] * u

  return pl.pallas_call(
      body,
      grid=(n // bn,),
      in_specs=[
          pl.BlockSpec((bn, d_in), lambda i: (i, 0)),
          pl.BlockSpec((bn, 1), lambda i: (i, 0)),
          pl.BlockSpec(W1.shape, lambda i: (0, 0)),
          pl.BlockSpec(Wve.shape, lambda i: (0, 0)),
          pl.BlockSpec((1, de), lambda i: (0, 0)),
          pl.BlockSpec((1, de), lambda i: (0, 0)),
      ],
      out_specs=pl.BlockSpec((bn, de), lambda i: (i, 0)),
      out_shape=jax.ShapeDtypeStruct((n, de), jnp.float32),
  )(vfeat, vrw, W1, Wve, b1, bve)


def _tc_phase2(p0, p1, ers, erw, Wev, bev):
  m, de = p0.shape
  dv = Wev.shape[1]
  bm = 2000

  def body(a, b, ersr, erwr, wev, bevr, o):
    fe = (a[...] + b[...]) / ersr[...]
    o[...] = erwr[...] * (
        jnp.dot(fe, wev[...], preferred_element_type=jnp.float32) + bevr[...])

  return pl.pallas_call(
      body,
      grid=(m // bm,),
      in_specs=[
          pl.BlockSpec((bm, de), lambda i: (i, 0)),
          pl.BlockSpec((bm, de), lambda i: (i, 0)),
          pl.BlockSpec((bm, 1), lambda i: (i, 0)),
          pl.BlockSpec((bm, 1), lambda i: (i, 0)),
          pl.BlockSpec(Wev.shape, lambda i: (0, 0)),
          pl.BlockSpec((1, dv), lambda i: (0, 0)),
      ],
      out_specs=pl.BlockSpec((bm, dv), lambda i: (i, 0)),
      out_shape=jax.ShapeDtypeStruct((m, dv), jnp.float32),
  )(p0, p1, ers, erw, Wev, bev)


def _tc_finish(q0, q1, vrs):
  n, dv = q0.shape
  bn = 2000

  def body(a, b, vr, o):
    o[...] = (a[...] + b[...]) / vr[...]

  return pl.pallas_call(
      body,
      grid=(n // bn,),
      in_specs=[
          pl.BlockSpec((bn, dv), lambda i: (i, 0)),
          pl.BlockSpec((bn, dv), lambda i: (i, 0)),
          pl.BlockSpec((bn, 1), lambda i: (i, 0)),
      ],
      out_specs=pl.BlockSpec((bn, dv), lambda i: (i, 0)),
      out_shape=jax.ShapeDtypeStruct((n, dv), jnp.float32),
  )(q0, q1, vrs)


def kernel(vfeat, efeat, v_reg_weight, v_reg_sum, e_reg_weight, e_reg_sum,
           in_edge_index, W1, b1, Wve, bve, Wev, bev):
  n = vfeat.shape[0]
  m = e_reg_sum.shape[0]
  e = in_edge_index.shape[1]
  nchunk = e // (NW * CH)

  idx = in_edge_index.astype(jnp.int32)
  src = idx[0].reshape(NW, nchunk, CH)
  dst = idx[1].reshape(NW, nchunk, CH)
  zeros = jnp.zeros((n, vfeat.shape[1]), jnp.float32)

  x = _tc_phase1(vfeat, v_reg_weight, W1, Wve, b1[None], bve[None])
  part_e = _sc_segsum(x, src, dst, zeros)
  y = _tc_phase2(part_e[:m], part_e[m:], e_reg_sum, e_reg_weight, Wev, bev[None])
  part_v = _sc_segsum(y, dst, src, zeros)
  return _tc_finish(part_v[:n], part_v[n:], v_reg_sum)


# speedup vs baseline: 32.5910x; 32.5910x over previous
import functools

import jax
import jax.numpy as jnp
from jax import lax
from jax.experimental import pallas as pl
from jax.experimental.pallas import tpu as pltpu
from jax.experimental.pallas import tpu_sc as plsc

NC = 2
NS = 16
CH = 125


def _sc_segsum(table2, gidx, sidx, zeros):
  _, R, DH = table2.shape
  nchunk = gidx.shape[1]
  rpt = (R // NS) // 8 * 8
  tail = R - NS * rpt
  mesh = plsc.VectorSubcoreMesh(core_axis_name="c", subcore_axis_name="s",
                                num_cores=NC, num_subcores=NS)

  @functools.partial(
      pl.kernel,
      out_type=jax.ShapeDtypeStruct((NC * R, DH), jnp.float32),
      mesh=mesh,
      scratch_types=[
          pltpu.VMEM((nchunk, CH), jnp.int32),
          pltpu.VMEM((nchunk, CH), jnp.int32),
          pltpu.VMEM((CH, DH), jnp.float32),
          pltpu.VMEM((CH, DH), jnp.float32),
          pltpu.VMEM_SHARED((R, DH), jnp.float32),
          pltpu.SemaphoreType.DMA,
          pltpu.SemaphoreType.DMA,
      ],
      compiler_params=pltpu.CompilerParams(use_tc_tiling_on_sc=False),
  )
  def k(table_hbm, gidx_hbm, sidx_hbm, zeros_hbm, part_hbm,
        gidx_v, sidx_v, rows0, rows1, acc, sem0, sem1):
    c = lax.axis_index("c")
    s = lax.axis_index("s")
    tbl = table_hbm.at[c]
    pltpu.sync_copy(zeros_hbm.at[pl.ds(s * rpt, rpt)], acc.at[pl.ds(s * rpt, rpt)])

    @pl.when(s == 0)
    def _():
      pltpu.sync_copy(zeros_hbm.at[pl.ds(NS * rpt, tail)],
                      acc.at[pl.ds(NS * rpt, tail)])

    pltpu.sync_copy(gidx_hbm.at[s], gidx_v)
    pltpu.sync_copy(sidx_hbm.at[s], sidx_v)
    plsc.subcore_barrier()

    pltpu.make_async_copy(tbl.at[gidx_v.at[0]], rows0, sem0).start()

    @pl.loop(0, nchunk // 2)
    def _(kk):
      g0 = 2 * kk
      pltpu.make_async_copy(tbl.at[gidx_v.at[g0 + 1]], rows1, sem1).start()
      pltpu.make_async_copy(tbl.at[gidx_v.at[g0]], rows0, sem0).wait()
      pltpu.sync_copy(rows0, acc.at[sidx_v.at[g0]], add=True)

      @pl.when(g0 + 2 < nchunk)
      def _():
        pltpu.make_async_copy(tbl.at[gidx_v.at[g0 + 2]], rows0, sem0).start()

      pltpu.make_async_copy(tbl.at[gidx_v.at[g0 + 1]], rows1, sem1).wait()
      pltpu.sync_copy(rows1, acc.at[sidx_v.at[g0 + 1]], add=True)

    if nchunk % 2 == 1:
      pltpu.make_async_copy(tbl.at[gidx_v.at[nchunk - 1]], rows0, sem0).wait()
      pltpu.sync_copy(rows0, acc.at[sidx_v.at[nchunk - 1]], add=True)

    plsc.subcore_barrier()
    pltpu.sync_copy(acc.at[pl.ds(s * rpt, rpt)],
                    part_hbm.at[pl.ds(c * R + s * rpt, rpt)])

    @pl.when(s == 0)
    def _():
      pltpu.sync_copy(acc.at[pl.ds(NS * rpt, tail)],
                      part_hbm.at[pl.ds(c * R + NS * rpt, tail)])

  return k(table2, gidx, sidx, zeros)


def _tc_phase1(vfeat, vrw, W1, Wve, b1, bve):
  n, d_in = vfeat.shape
  de = Wve.shape[1]
  dh = de // 2
  bn = 2000

  def body(vf, wr, w1, wv, b1r, bver, o):
    t = jnp.dot(vf[...], w1[...], preferred_element_type=jnp.float32)
    bias = jnp.dot(b1r[...], wv[...], preferred_element_type=jnp.float32) + bver[...]
    u = wr[...] * (jnp.dot(t, wv[...], preferred_element_type=jnp.float32) + bias)
    o[0] = u[:, :dh]
    o[1] = u[:, dh:]

  return pl.pallas_call(
      body,
      grid=(n // bn,),
      in_specs=[
          pl.BlockSpec((bn, d_in), lambda i: (i, 0)),
          pl.BlockSpec((bn, 1), lambda i: (i, 0)),
          pl.BlockSpec(W1.shape, lambda i: (0, 0)),
          pl.BlockSpec(Wve.shape, lambda i: (0, 0)),
          pl.BlockSpec((1, de), lambda i: (0, 0)),
          pl.BlockSpec((1, de), lambda i: (0, 0)),
      ],
      out_specs=pl.BlockSpec((2, bn, dh), lambda i: (0, i, 0)),
      out_shape=jax.ShapeDtypeStruct((2, n, dh), jnp.float32),
  )(vfeat, vrw, W1, Wve, b1, bve)


def _tc_phase2(p0, p1, ers, erw, Wev, bev):
  m, dh = p0.shape
  dv = Wev.shape[1]
  dvh = dv // 2
  bm = 2000

  def body(a, b, ersr, erwr, wev, bevr, o):
    f0 = a[...] / ersr[...]
    f1 = b[...] / ersr[...]
    u = jnp.dot(f0, wev[:dh], preferred_element_type=jnp.float32)
    u += jnp.dot(f1, wev[dh:], preferred_element_type=jnp.float32)
    y = erwr[...] * (u + bevr[...])
    o[0] = y[:, :dvh]
    o[1] = y[:, dvh:]

  return pl.pallas_call(
      body,
      grid=(m // bm,),
      in_specs=[
          pl.BlockSpec((bm, dh), lambda i: (i, 0)),
          pl.BlockSpec((bm, dh), lambda i: (i, 0)),
          pl.BlockSpec((bm, 1), lambda i: (i, 0)),
          pl.BlockSpec((bm, 1), lambda i: (i, 0)),
          pl.BlockSpec(Wev.shape, lambda i: (0, 0)),
          pl.BlockSpec((1, dv), lambda i: (0, 0)),
      ],
      out_specs=pl.BlockSpec((2, bm, dvh), lambda i: (0, i, 0)),
      out_shape=jax.ShapeDtypeStruct((2, m, dvh), jnp.float32),
  )(p0, p1, ers, erw, Wev, bev)


def _tc_finish(q0, q1, vrs):
  n, dvh = q0.shape
  bn = 2000

  def body(a, b, vr, o):
    o[...] = jnp.concatenate([a[...], b[...]], axis=1) / vr[...]

  return pl.pallas_call(
      body,
      grid=(n // bn,),
      in_specs=[
          pl.BlockSpec((bn, dvh), lambda i: (i, 0)),
          pl.BlockSpec((bn, dvh), lambda i: (i, 0)),
          pl.BlockSpec((bn, 1), lambda i: (i, 0)),
      ],
      out_specs=pl.BlockSpec((bn, 2 * dvh), lambda i: (i, 0)),
      out_shape=jax.ShapeDtypeStruct((n, 2 * dvh), jnp.float32),
  )(q0, q1, vrs)


def kernel(vfeat, efeat, v_reg_weight, v_reg_sum, e_reg_weight, e_reg_sum,
           in_edge_index, W1, b1, Wve, bve, Wev, bev):
  n = vfeat.shape[0]
  m = e_reg_sum.shape[0]
  e = in_edge_index.shape[1]
  nchunk = e // (NS * CH)

  idx = in_edge_index.astype(jnp.int32)
  src = idx[0].reshape(NS, nchunk, CH)
  dst = idx[1].reshape(NS, nchunk, CH)
  zeros = jnp.zeros((n, vfeat.shape[1] // 2), jnp.float32)

  x2 = _tc_phase1(vfeat, v_reg_weight, W1, Wve, b1[None], bve[None])
  part_e = _sc_segsum(x2, src, dst, zeros)
  y2 = _tc_phase2(part_e[:m], part_e[m:], e_reg_sum, e_reg_weight, Wev, bev[None])
  part_v = _sc_segsum(y2, dst, src, zeros)
  return _tc_finish(part_v[:n], part_v[n:], v_reg_sum)
